# baseline (speedup 1.0000x reference)
import functools

import jax
import jax.numpy as jnp
from jax import lax
from jax.experimental import pallas as pl
from jax.experimental.pallas import tpu as pltpu
from jax.experimental.pallas import tpu_sc as plsc

D_MODEL = 64
VPAD = 128
SEQ = 200
LANES = 16
FULL_GROUPS = SEQ // LANES
TAIL = SEQ - LANES


@functools.lru_cache(maxsize=None)
def _make_kernel(NB):
    info = plsc.get_sparse_core_info()
    nc, ns = info.num_cores, info.num_subcores
    nw = nc * ns
    per_w = NB // nw
    assert NB % nw == 0 and per_w % 4 == 0

    mesh = plsc.VectorSubcoreMesh(core_axis_name="c", subcore_axis_name="s")

    @functools.partial(
        pl.kernel,
        mesh=mesh,
        out_type=jax.ShapeDtypeStruct((NB, SEQ, D_MODEL), jnp.float32),
        scratch_types=[
            pltpu.VMEM((2 * SEQ,), jnp.int32),
            pltpu.VMEM((2 * SEQ,), jnp.int32),
            pltpu.VMEM((2, SEQ, D_MODEL), jnp.float32),
            pltpu.VMEM((2, SEQ, D_MODEL), jnp.float32),
            pltpu.VMEM((VPAD * VPAD,), jnp.float32),
            pltpu.SemaphoreType.DMA,
            pltpu.SemaphoreType.DMA,
            pltpu.SemaphoreType.DMA,
            pltpu.SemaphoreType.DMA,
        ],
        compiler_params=pltpu.CompilerParams(needs_layout_passes=False),
    )
    def k(x_hbm, table_hbm, out_hbm, idx_v0, idx_v1, rows_v0, rows_v1,
          table_v, isem, gsem, osem0, osem1):
        wid = lax.axis_index("s") * nc + lax.axis_index("c")
        base = wid * per_w

        idx_bufs = (idx_v0, idx_v1)
        rows_bufs = (rows_v0, rows_v1)
        osems = (osem0, osem1)

        pltpu.async_copy(table_hbm, table_v, gsem)
        pltpu.async_copy(
            x_hbm.at[pl.ds(base * SEQ, 2 * SEQ)], idx_v0, isem)
        pltpu.make_async_copy(table_hbm, table_v, gsem).wait()

        iota = lax.iota(jnp.int32, LANES)

        def expand(idx_b, rows_b, bb, row0):
            idxs = idx_b[pl.ds(bb * SEQ + row0, LANES)]
            srcb = idxs * VPAD
            for kk in range(LANES):
                lane = jnp.full((LANES, 1), kk, jnp.int32)
                splat = lax.gather(
                    srcb, lane,
                    dimension_numbers=lax.GatherDimensionNumbers(
                        offset_dims=(), collapsed_slice_dims=(0,),
                        start_index_map=(0,)),
                    slice_sizes=(1,),
                    mode=lax.GatherScatterMode.PROMISE_IN_BOUNDS)
                for q in range(0, D_MODEL, LANES):
                    vals = plsc.load_gather(table_v, [splat + (iota + q)])
                    rows_b[bb, row0 + kk, pl.ds(q, LANES)] = vals

        def body(s, carry):
            for b in range(2):
                g = 2 * s + b
                bat = base + 2 * g
                idx_b, rows_b, osem_b = idx_bufs[b], rows_bufs[b], osems[b]
                pltpu.make_async_copy(
                    x_hbm.at[pl.ds(bat * SEQ, 2 * SEQ)], idx_b, isem).wait()
                nxt = jnp.minimum(2 * (g + 1), per_w - 2)
                pltpu.async_copy(
                    x_hbm.at[pl.ds((base + nxt) * SEQ, 2 * SEQ)],
                    idx_bufs[1 - b], isem)

                @pl.when(s > 0)
                def _():
                    pltpu.make_async_copy(
                        rows_b, out_hbm.at[pl.ds(0, 2)], osem_b).wait()

                for bb in range(2):
                    @plsc.parallel_loop(0, FULL_GROUPS, step=1, unroll=1)
                    def group(i, bb=bb):
                        expand(idx_b, rows_b, bb, i * LANES)

                    expand(idx_b, rows_b, bb, TAIL)

                pltpu.async_copy(
                    rows_b, out_hbm.at[pl.ds(bat, 2)], osem_b)
            return carry

        lax.fori_loop(0, per_w // 4, body, 0)

        pltpu.make_async_copy(
            rows_v0, out_hbm.at[pl.ds(0, 2)], osem0).wait()
        pltpu.make_async_copy(
            rows_v1, out_hbm.at[pl.ds(0, 2)], osem1).wait()
        pltpu.make_async_copy(
            x_hbm.at[pl.ds(base * SEQ, 2 * SEQ)], idx_v0, isem).wait()

    return k


def kernel(x, table):
    nb, s = x.shape
    x1d = x.reshape(nb * s).astype(jnp.int32)
    tpad = jnp.zeros((VPAD, VPAD), jnp.float32)
    tpad = tpad.at[:table.shape[0], :D_MODEL].set(table).reshape(VPAD * VPAD)
    return _make_kernel(nb)(x1d, tpad)


# speedup vs baseline: 1.5176x; 1.5176x over previous
import functools

import jax
import jax.numpy as jnp
from jax import lax
from jax.experimental import pallas as pl
from jax.experimental.pallas import tpu as pltpu
from jax.experimental.pallas import tpu_sc as plsc

D_MODEL = 64
VPAD = 128
CHUNK = 400
LANES = 16
GROUPS = CHUNK // LANES


@functools.lru_cache(maxsize=None)
def _make_kernel(B):
    info = plsc.get_sparse_core_info()
    nc, ns = info.num_cores, info.num_subcores
    nw = nc * ns
    per_w = B // nw
    n_chunks = per_w // CHUNK
    assert per_w % CHUNK == 0 and n_chunks % 2 == 0

    mesh = plsc.VectorSubcoreMesh(core_axis_name="c", subcore_axis_name="s")

    @functools.partial(
        pl.kernel,
        mesh=mesh,
        out_type=jax.ShapeDtypeStruct((B, D_MODEL), jnp.float32),
        scratch_types=[
            pltpu.VMEM((CHUNK,), jnp.int32),
            pltpu.VMEM((CHUNK,), jnp.int32),
            pltpu.VMEM((CHUNK, D_MODEL), jnp.float32),
            pltpu.VMEM((CHUNK, D_MODEL), jnp.float32),
            pltpu.VMEM((VPAD * VPAD,), jnp.float32),
            pltpu.SemaphoreType.DMA,
            pltpu.SemaphoreType.DMA,
            pltpu.SemaphoreType.DMA,
            pltpu.SemaphoreType.DMA,
        ],
        compiler_params=pltpu.CompilerParams(needs_layout_passes=False),
    )
    def k(x_hbm, table_hbm, out_hbm, idx_v0, idx_v1, rows_v0, rows_v1,
          table_v, isem, gsem, osem0, osem1):
        wid = lax.axis_index("s") * nc + lax.axis_index("c")
        base = wid * per_w

        idx_bufs = (idx_v0, idx_v1)
        rows_bufs = (rows_v0, rows_v1)
        osems = (osem0, osem1)

        pltpu.async_copy(table_hbm, table_v, gsem)
        pltpu.async_copy(x_hbm.at[pl.ds(base, CHUNK)], idx_v0, isem)
        pltpu.make_async_copy(table_hbm, table_v, gsem).wait()

        iota = lax.iota(jnp.int32, LANES)

        def body(s, carry):
            for b in range(2):
                g = 2 * s + b
                off = base + g * CHUNK
                idx_b, rows_b, osem_b = idx_bufs[b], rows_bufs[b], osems[b]
                pltpu.make_async_copy(
                    x_hbm.at[pl.ds(off, CHUNK)], idx_b, isem).wait()
                nxt = jnp.minimum(g + 1, n_chunks - 1)
                pltpu.async_copy(
                    x_hbm.at[pl.ds(base + nxt * CHUNK, CHUNK)],
                    idx_bufs[1 - b], isem)

                @pl.when(s > 0)
                def _():
                    pltpu.make_async_copy(
                        rows_b, out_hbm.at[pl.ds(0, CHUNK)], osem_b).wait()

                @plsc.parallel_loop(0, GROUPS, step=1, unroll=1)
                def group(i):
                    idxs = idx_b[pl.ds(i * LANES, LANES)]
                    srcb = idxs * VPAD
                    for kk in range(LANES):
                        lane = jnp.full((LANES, 1), kk, jnp.int32)
                        splat = lax.gather(
                            srcb, lane,
                            dimension_numbers=lax.GatherDimensionNumbers(
                                offset_dims=(), collapsed_slice_dims=(0,),
                                start_index_map=(0,)),
                            slice_sizes=(1,),
                            mode=lax.GatherScatterMode.PROMISE_IN_BOUNDS)
                        rr = i * LANES + kk
                        for q in range(0, D_MODEL, LANES):
                            vals = plsc.load_gather(
                                table_v, [splat + (iota + q)])
                            rows_b[rr, pl.ds(q, LANES)] = vals

                pltpu.async_copy(
                    rows_b, out_hbm.at[pl.ds(off, CHUNK)], osem_b)
            return carry

        lax.fori_loop(0, n_chunks // 2, body, 0)

        pltpu.make_async_copy(
            rows_v0, out_hbm.at[pl.ds(0, CHUNK)], osem0).wait()
        pltpu.make_async_copy(
            rows_v1, out_hbm.at[pl.ds(0, CHUNK)], osem1).wait()
        pltpu.make_async_copy(
            x_hbm.at[pl.ds(base, CHUNK)], idx_v0, isem).wait()

    return k


def kernel(x, table):
    b, s = x.shape
    total = b * s
    x1d = x.reshape(total).astype(jnp.int32)
    tpad = jnp.zeros((VPAD, VPAD), jnp.float32)
    tpad = tpad.at[:table.shape[0], :D_MODEL].set(table).reshape(VPAD * VPAD)
    out = _make_kernel(total)(x1d, tpad)
    return out.reshape(b, s, D_MODEL)
